# Initial kernel scaffold; baseline (speedup 1.0000x reference)
import jax
import jax.numpy as jnp
from jax.experimental import pallas as pl


def kernel(pred_logits, pred_points, tgt_points, tgt_ids):
    raise NotImplementedError("write your pallas kernel here")


# speedup vs baseline: 20.2715x; 20.2715x over previous
import jax
import jax.numpy as jnp
from jax.experimental import pallas as pl

_BR = 256
_K = 5


def _cost_body(q_ref, l_ref, t_ref, c_ref, o_ref):
    nt = t_ref.shape[1]
    qx = q_ref[:, 0:1]
    qy = q_ref[:, 1:2]
    tx = t_ref[0:1, :]
    ty = t_ref[1:2, :]
    dx = qx - tx
    dy = qy - ty
    s2 = dx * dx + dy * dy
    d = jnp.sqrt(s2)

    iota = jax.lax.broadcasted_iota(jnp.int32, s2.shape, 1)
    cur = s2
    total = jnp.zeros((s2.shape[0], 1), jnp.float32)
    for i in range(_K):
        m = jnp.min(cur, axis=1, keepdims=True)
        total = total + jnp.sqrt(m)
        if i < _K - 1:
            pos = jnp.min(jnp.where(cur <= m, iota, nt), axis=1,
                          keepdims=True)
            cur = jnp.where(iota == pos, jnp.float32(jnp.inf), cur)
    delta = total * (1.0 / _K)

    p0 = jax.nn.sigmoid(l_ref[:, 0:1] - l_ref[:, 1:2])
    tcls = c_ref[0:1, :]
    cls_cost = p0 + tcls * (1.0 - 2.0 * p0)

    w = jnp.exp(s2 * (-1.0 / 50.0))
    cost_point = jnp.where(d < delta, d * w, d)
    o_ref[:, :] = cost_point - cls_cost


@jax.jit
def kernel(pred_logits, pred_points, tgt_points, tgt_ids):
    bs, nq, _ = pred_logits.shape
    nt = tgt_points.shape[0]
    nq_flat = bs * nq
    q = pred_points.reshape(nq_flat, 2)
    logits = pred_logits.reshape(nq_flat, 2)
    t_t = tgt_points.T
    cls = tgt_ids.astype(jnp.float32).reshape(1, nt)

    out = pl.pallas_call(
        _cost_body,
        grid=(nq_flat // _BR,),
        in_specs=[
            pl.BlockSpec((_BR, 2), lambda i: (i, 0)),
            pl.BlockSpec((_BR, 2), lambda i: (i, 0)),
            pl.BlockSpec((2, nt), lambda i: (0, 0)),
            pl.BlockSpec((1, nt), lambda i: (0, 0)),
        ],
        out_specs=pl.BlockSpec((_BR, nt), lambda i: (i, 0)),
        out_shape=jax.ShapeDtypeStruct((nq_flat, nt), jnp.float32),
    )(q, logits, t_t, cls)
    return out.reshape(bs, nq, nt)


# speedup vs baseline: 20.4947x; 1.0110x over previous
import jax
import jax.numpy as jnp
from jax.experimental import pallas as pl

_BR = 256
_K = 5


def _cost_body(q_ref, l_ref, t_ref, c_ref, o_ref):
    nt = t_ref.shape[1]
    qx = q_ref[:, 0:1]
    qy = q_ref[:, 1:2]
    tx = t_ref[0:1, :]
    ty = t_ref[1:2, :]
    dx = qx - tx
    dy = qy - ty
    s2 = dx * dx + dy * dy
    d = jnp.sqrt(s2)

    iota = jax.lax.broadcasted_iota(jnp.int32, s2.shape, 1)
    cur = d
    total = jnp.zeros((s2.shape[0], 1), jnp.float32)
    for i in range(_K):
        m = jnp.min(cur, axis=1, keepdims=True)
        total = total + m
        if i < _K - 1:
            pos = jnp.min(jnp.where(cur <= m, iota, nt), axis=1,
                          keepdims=True)
            cur = jnp.where(iota == pos, jnp.float32(jnp.inf), cur)
    delta = total / jnp.float32(_K)

    p0 = jax.nn.sigmoid(l_ref[:, 0:1] - l_ref[:, 1:2])
    tcls = c_ref[0:1, :]
    cls_cost = p0 + tcls * (1.0 - 2.0 * p0)

    w = jnp.exp(s2 * (-1.0 / 50.0))
    cost_point = jnp.where(d < delta, d * w, d)
    o_ref[:, :] = cost_point - cls_cost


@jax.jit
def kernel(pred_logits, pred_points, tgt_points, tgt_ids):
    bs, nq, _ = pred_logits.shape
    nt = tgt_points.shape[0]
    nq_flat = bs * nq
    q = pred_points.reshape(nq_flat, 2)
    logits = pred_logits.reshape(nq_flat, 2)
    t_t = tgt_points.T
    cls = tgt_ids.astype(jnp.float32).reshape(1, nt)

    out = pl.pallas_call(
        _cost_body,
        grid=(nq_flat // _BR,),
        in_specs=[
            pl.BlockSpec((_BR, 2), lambda i: (i, 0)),
            pl.BlockSpec((_BR, 2), lambda i: (i, 0)),
            pl.BlockSpec((2, nt), lambda i: (0, 0)),
            pl.BlockSpec((1, nt), lambda i: (0, 0)),
        ],
        out_specs=pl.BlockSpec((_BR, nt), lambda i: (i, 0)),
        out_shape=jax.ShapeDtypeStruct((nq_flat, nt), jnp.float32),
    )(q, logits, t_t, cls)
    return out.reshape(bs, nq, nt)
